# baseline (speedup 1.0000x reference)
import functools

import jax
import jax.numpy as jnp
from jax import lax
from jax.experimental import pallas as pl
from jax.experimental.pallas import tpu as pltpu
from jax.experimental.pallas import tpu_sc as plsc

VOCAB = 1000000
EMBED_DIM = 64
BATCH = 4096
HIST = 200
B = BATCH * HIST

_INFO = plsc.get_sparse_core_info()
NC = _INFO.num_cores
NS = _INFO.num_subcores
NW = NC * NS

BLKW = 384
NBLK = VOCAB // BLKW
TAIL = VOCAB - NBLK * BLKW
DENSE_ROWS = VOCAB // 2 + 32
BLK_PER_W = (NBLK + NW - 1) // NW


@functools.partial(
    pl.kernel,
    out_type=jax.ShapeDtypeStruct((DENSE_ROWS, 128), jnp.float32),
    mesh=plsc.VectorSubcoreMesh(core_axis_name="c", subcore_axis_name="s"),
    scratch_types=[
        pltpu.VMEM((2, 64, BLKW), jnp.float32),
        pltpu.VMEM((2, BLKW // 2, 128), jnp.float32),
        pltpu.VMEM((64, 64), jnp.float32),
        pltpu.SemaphoreType.DMA((2,)),
        pltpu.SemaphoreType.DMA((2,)),
    ],
    compiler_params=pltpu.CompilerParams(
        use_tc_tiling_on_sc=True, needs_layout_passes=False),
)
def _table_transpose(tableT_hbm, tailT_hbm, dense_hbm, src_v, dst_v, tail_v,
                     sem_i, sem_o):
    w = lax.axis_index("s") * NC + lax.axis_index("c")
    j0 = w * BLK_PER_W

    iota = lax.iota(jnp.int32, 16)

    dvs = [16 * g + iota for g in range(4)]

    def start_in(jj, b):
        j = j0 + jj

        @pl.when((jj < BLK_PER_W) & (j < NBLK))
        def _():
            pltpu.async_copy(
                tableT_hbm.at[:, pl.ds(j * BLKW, BLKW)], src_v.at[b],
                sem_i.at[b])

    def wait_in(b):
        pltpu.make_async_copy(
            tableT_hbm.at[:, pl.ds(0, BLKW)], src_v.at[b], sem_i.at[b]).wait()

    def wait_out(b):
        pltpu.make_async_copy(
            dst_v.at[b], dense_hbm.at[pl.ds(0, BLKW // 2)], sem_o.at[b]).wait()

    def transpose(src_ref, b, width):
        @plsc.parallel_loop(0, width, unroll=8)
        def diag(c0):
            cv = c0 + iota
            cv = jnp.where(cv >= width, cv - width, cv)
            qv = lax.shift_right_logical(cv, 1)
            c2v = lax.shift_left(jnp.bitwise_and(cv, 1), 6)
            for g in range(4):
                vals = plsc.load_gather(src_ref, [dvs[g], cv])
                plsc.store_scatter(dst_v.at[b], [qv, c2v + dvs[g]], vals)

    start_in(0, 0)
    start_in(1, 1)

    def outer(g, carry):
        for b in range(2):
            jj = g * 2 + b
            j = j0 + jj

            @pl.when((jj < BLK_PER_W) & (j < NBLK))
            def _():
                wait_in(b)

                @pl.when(jj >= 2)
                def _():
                    wait_out(b)

                transpose(src_v.at[b], b, BLKW)
                start_in(jj + 2, b)
                pltpu.async_copy(
                    dst_v.at[b],
                    dense_hbm.at[pl.ds(j * (BLKW // 2), BLKW // 2)],
                    sem_o.at[b])
        return carry

    lax.fori_loop(0, (BLK_PER_W + 1) // 2, outer, 0)

    nv = jnp.minimum(NBLK - j0, BLK_PER_W)
    for b in range(2):
        @pl.when(nv > b)
        def _():
            wait_out(b)

    @pl.when(w == NW - 1)
    def _():
        pltpu.sync_copy(tailT_hbm, tail_v)
        transpose(tail_v, 0, TAIL)
        pltpu.sync_copy(
            dst_v.at[0, pl.ds(0, TAIL // 2), :],
            dense_hbm.at[pl.ds(NBLK * (BLKW // 2), TAIL // 2)])


BPW = B // NW


@functools.partial(
    pl.kernel,
    out_type=jax.ShapeDtypeStruct((HIST, 8, 32, 8, 128), jnp.float32),
    mesh=plsc.VectorSubcoreMesh(core_axis_name="c", subcore_axis_name="s"),
    scratch_types=[
        pltpu.VMEM((BPW,), jnp.int32),
        pltpu.VMEM((HIST, 128), jnp.int32),
        pltpu.VMEM((3, 128, EMBED_DIM), jnp.float32),
        pltpu.VMEM((3, 8, 8, 128), jnp.float32),
        pltpu.SemaphoreType.DMA((3,)),
        pltpu.SemaphoreType.DMA((3,)),
    ],
    compiler_params=pltpu.CompilerParams(
        use_tc_tiling_on_sc=False, needs_layout_passes=False),
)
def _embed_gather(doc_hbm, table_hbm, out_hbm, doc_v, idxh_v, rows_v, tile_v,
                  sem_g, sem_o):
    w = lax.axis_index("s") * NC + lax.axis_index("c")
    iota = lax.iota(jnp.int32, 16)

    pltpu.sync_copy(doc_hbm.at[pl.ds(w * BPW, BPW)], doc_v)

    @plsc.parallel_loop(0, HIST, unroll=4)
    def docT(h0):
        hv = h0 + iota
        hv = jnp.where(hv >= HIST, hv - HIST, hv)
        for g in range(8):
            bv = 16 * g + iota
            vals = plsc.load_gather(doc_v, [bv * HIST + hv])
            plsc.store_scatter(idxh_v, [hv, bv], vals)

    def start_gather(h, b):
        pltpu.async_copy(
            table_hbm.at[idxh_v.at[h]], rows_v.at[b], sem_g.at[b])

    def wait_gather(b):
        pltpu.make_async_copy(
            table_hbm.at[idxh_v.at[0]], rows_v.at[b], sem_g.at[b]).wait()

    def wait_out(b):
        pltpu.make_async_copy(
            tile_v.at[b], out_hbm.at[0, :, 0], sem_o.at[b]).wait()

    def transpose_rows(b):
        @plsc.parallel_loop(0, EMBED_DIM, unroll=4)
        def diag(d0):
            dv = d0 + iota
            dv = jnp.where(dv >= EMBED_DIM, dv - EMBED_DIM, dv)
            dtv = lax.shift_right_logical(dv, 3)
            dsv = jnp.bitwise_and(dv, 7)
            for g in range(8):
                bv = 16 * g + iota
                vals = plsc.load_gather(rows_v.at[b], [bv, dv])
                plsc.store_scatter(tile_v.at[b], [dtv, dsv, bv], vals)

    for b in range(3):
        start_gather(b, b)

    def outer(g2, carry):
        for b in range(3):
            h = g2 * 3 + b

            @pl.when(h < HIST)
            def _():
                wait_gather(b)

                @pl.when(h >= 3)
                def _():
                    wait_out(b)

                transpose_rows(b)

                @pl.when(h + 3 < HIST)
                def _():
                    start_gather(h + 3, b)

                pltpu.async_copy(
                    tile_v.at[b], out_hbm.at[h, :, w], sem_o.at[b])
        return carry

    lax.fori_loop(0, (HIST + 2) // 3, outer, 0)

    for b in range(3):
        wait_out(b)


def kernel(doc, table):
    flat = doc.reshape(B).astype(jnp.int32)
    dense = _table_transpose(table.T, table.T[:, VOCAB - TAIL:])
    dense_rows = dense.reshape(DENSE_ROWS * 2, EMBED_DIM)
    out5 = _embed_gather(flat, dense_rows)
    return out5.transpose(2, 4, 0, 1, 3).reshape(BATCH, HIST, EMBED_DIM)
